# Initial kernel scaffold; baseline (speedup 1.0000x reference)
import functools

import jax
import jax.numpy as jnp
from jax import lax
from jax.experimental import pallas as pl
from jax.experimental.pallas import tpu as pltpu
from jax.experimental.pallas import tpu_sc as plsc

NC = 2
NS = 16
NT = NC * NS
CH = 128
RB = 512


def _sc_degree(dst2d, n_pad):
    d = 16
    n_chunks = dst2d.shape[0]
    cpt = n_chunks // NT
    rpt = n_pad // NS

    mesh = plsc.VectorSubcoreMesh(core_axis_name="c", subcore_axis_name="s")

    @functools.partial(
        pl.kernel,
        out_type=jax.ShapeDtypeStruct((NC * n_pad, d), jnp.float32),
        mesh=mesh,
        scratch_types=[
            pltpu.VMEM((CH,), jnp.int32),
            pltpu.VMEM((CH, d), jnp.float32),
            pltpu.VMEM_SHARED((n_pad, d), jnp.float32),
        ],
    )
    def run(dst_hbm, out_hbm, didx, buf, accum):
        c = lax.axis_index("c")
        s = lax.axis_index("s")
        wid = c * NS + s

        @pl.loop(0, CH)
        def _(r):
            buf[r, pl.ds(0, 16)] = jnp.zeros((16,), jnp.float32)

        @pl.loop(0, rpt, step=CH)
        def _(r):
            pltpu.sync_copy(buf, accum.at[pl.ds(s * rpt + r, CH)])

        @pl.loop(0, CH)
        def _(r):
            buf[r, pl.ds(0, 16)] = jnp.ones((16,), jnp.float32)

        plsc.subcore_barrier()

        @pl.loop(0, cpt)
        def _(i):
            ci = wid * cpt + i
            pltpu.sync_copy(dst_hbm.at[ci], didx)
            pltpu.sync_copy(buf, accum.at[didx], add=True)

        plsc.subcore_barrier()
        pltpu.sync_copy(
            accum.at[pl.ds(s * rpt, rpt)],
            out_hbm.at[pl.ds(c * n_pad + s * rpt, rpt)],
        )

    return run(dst2d)


def _sc_segment_sum(y, src2d, dst2d, n_pad):
    d = y.shape[1]
    n_chunks = src2d.shape[0]
    cpt = n_chunks // NT
    rpt = n_pad // NS

    mesh = plsc.VectorSubcoreMesh(core_axis_name="c", subcore_axis_name="s")

    @functools.partial(
        pl.kernel,
        out_type=jax.ShapeDtypeStruct((NC * n_pad, d), jnp.float32),
        mesh=mesh,
        scratch_types=[
            pltpu.VMEM((CH,), jnp.int32),
            pltpu.VMEM((CH,), jnp.int32),
            pltpu.VMEM((CH, d), jnp.float32),
            pltpu.VMEM_SHARED((n_pad, d), jnp.float32),
            pltpu.SemaphoreType.DMA,
        ],
    )
    def run(y_hbm, src_hbm, dst_hbm, out_hbm, sidx, didx, buf, accum, sem):
        c = lax.axis_index("c")
        s = lax.axis_index("s")
        wid = c * NS + s

        @pl.loop(0, CH)
        def _(r):
            @pl.loop(0, d, step=16)
            def _(col):
                buf[r, pl.ds(col, 16)] = jnp.zeros((16,), jnp.float32)

        @pl.loop(0, rpt, step=CH)
        def _(r):
            pltpu.sync_copy(buf, accum.at[pl.ds(s * rpt + r, CH)])

        plsc.subcore_barrier()

        @pl.loop(0, cpt)
        def _(i):
            ci = wid * cpt + i
            pltpu.sync_copy(src_hbm.at[ci], sidx)
            pltpu.sync_copy(dst_hbm.at[ci], didx)
            pltpu.async_copy(y_hbm.at[sidx], buf, sem).wait()
            pltpu.sync_copy(buf, accum.at[didx], add=True)

        plsc.subcore_barrier()
        pltpu.sync_copy(
            accum.at[pl.ds(s * rpt, rpt)],
            out_hbm.at[pl.ds(c * n_pad + s * rpt, rpt)],
        )

    return run(y, src2d, dst2d)


def _dinv(d0, d1):
    return lax.rsqrt(1.0 + d0[:, 0:1] + d1[:, 0:1])


def _mm_body(x_ref, w_ref, o_ref):
    o_ref[...] = jnp.dot(x_ref[...], w_ref[...],
                         preferred_element_type=jnp.float32,
                         precision=lax.Precision.HIGHEST)


def _tc_matmul(x_pad, w):
    n_pad, k = x_pad.shape
    m = w.shape[1]
    return pl.pallas_call(
        _mm_body,
        grid=(n_pad // RB,),
        in_specs=[pl.BlockSpec((RB, k), lambda i: (i, 0)),
                  pl.BlockSpec((k, m), lambda i: (0, 0))],
        out_specs=pl.BlockSpec((RB, m), lambda i: (i, 0)),
        out_shape=jax.ShapeDtypeStruct((n_pad, m), jnp.float32),
    )(x_pad, w)


def _scale_body(d0_ref, d1_ref, xw_ref, y_ref):
    y_ref[...] = xw_ref[...] * _dinv(d0_ref[...], d1_ref[...])


def _tc_scale(deg_p, xw):
    n_pad, dh = xw.shape
    nb = n_pad // RB
    return pl.pallas_call(
        _scale_body,
        grid=(nb,),
        in_specs=[pl.BlockSpec((RB, 16), lambda i: (i, 0)),
                  pl.BlockSpec((RB, 16), lambda i: (i + nb, 0)),
                  pl.BlockSpec((RB, dh), lambda i: (i, 0))],
        out_specs=pl.BlockSpec((RB, dh), lambda i: (i, 0)),
        out_shape=jax.ShapeDtypeStruct((n_pad, dh), jnp.float32),
    )(deg_p, deg_p, xw)


def _mid_body(d0, d1, a0, a1, xw1, b1, w2, xw2_o, y2_o):
    dinv = _dinv(d0[...], d1[...])
    h = (a0[...] + a1[...]) * dinv + xw1[...] * (dinv * dinv) + b1[...]
    h = jnp.maximum(h, 0.0)
    xw2 = jnp.dot(h, w2[...], preferred_element_type=jnp.float32,
                  precision=lax.Precision.HIGHEST)
    xw2_o[...] = xw2
    y2_o[...] = xw2 * dinv


def _tc_mid(deg_p, agg1_p, xw1, b1r, w2):
    n_pad, dh = xw1.shape
    do = w2.shape[1]
    nb = n_pad // RB
    return pl.pallas_call(
        _mid_body,
        grid=(nb,),
        in_specs=[pl.BlockSpec((RB, 16), lambda i: (i, 0)),
                  pl.BlockSpec((RB, 16), lambda i: (i + nb, 0)),
                  pl.BlockSpec((RB, dh), lambda i: (i, 0)),
                  pl.BlockSpec((RB, dh), lambda i: (i + nb, 0)),
                  pl.BlockSpec((RB, dh), lambda i: (i, 0)),
                  pl.BlockSpec((1, dh), lambda i: (0, 0)),
                  pl.BlockSpec((dh, do), lambda i: (0, 0))],
        out_specs=[pl.BlockSpec((RB, do), lambda i: (i, 0)),
                   pl.BlockSpec((RB, do), lambda i: (i, 0))],
        out_shape=[jax.ShapeDtypeStruct((n_pad, do), jnp.float32),
                   jax.ShapeDtypeStruct((n_pad, do), jnp.float32)],
    )(deg_p, deg_p, agg1_p, agg1_p, xw1, b1r, w2)


def _final_body(d0, d1, g0, g1, xw2, b2, o_ref):
    dinv = _dinv(d0[...], d1[...])
    o_ref[...] = (g0[...] + g1[...]) * dinv + xw2[...] * (dinv * dinv) + b2[...]


def _tc_final(deg_p, agg2_p, xw2, b2r):
    n_pad, do = xw2.shape
    nb = n_pad // RB
    return pl.pallas_call(
        _final_body,
        grid=(nb,),
        in_specs=[pl.BlockSpec((RB, 16), lambda i: (i, 0)),
                  pl.BlockSpec((RB, 16), lambda i: (i + nb, 0)),
                  pl.BlockSpec((RB, do), lambda i: (i, 0)),
                  pl.BlockSpec((RB, do), lambda i: (i + nb, 0)),
                  pl.BlockSpec((RB, do), lambda i: (i, 0)),
                  pl.BlockSpec((1, do), lambda i: (0, 0))],
        out_specs=pl.BlockSpec((RB, do), lambda i: (i, 0)),
        out_shape=jax.ShapeDtypeStruct((n_pad, do), jnp.float32),
    )(deg_p, deg_p, agg2_p, agg2_p, xw2, b2r)


def kernel(x, edge_index, W1, b1, W2, b2):
    n, d_in = x.shape
    e = edge_index.shape[1]

    blk = NS * CH
    n_pad = ((n + 1 + blk - 1) // blk) * blk
    epb = NT * CH
    e_pad = ((e + epb - 1) // epb) * epb

    x_pad = jnp.pad(x, ((0, n_pad - n), (0, 0)))
    pad_idx = jnp.full((e_pad - e,), n, dtype=jnp.int32)
    src2d = jnp.concatenate([edge_index[0], pad_idx]).reshape(e_pad // CH, CH)
    dst2d = jnp.concatenate([edge_index[1], pad_idx]).reshape(e_pad // CH, CH)

    deg_p = _sc_degree(dst2d, n_pad)
    xw1 = _tc_matmul(x_pad, W1)
    y1 = _tc_scale(deg_p, xw1)
    agg1_p = _sc_segment_sum(y1, src2d, dst2d, n_pad)
    xw2, y2 = _tc_mid(deg_p, agg1_p, xw1, b1.reshape(1, -1), W2)
    agg2_p = _sc_segment_sum(y2, src2d, dst2d, n_pad)
    out = _tc_final(deg_p, agg2_p, xw2, b2.reshape(1, -1))
    return out[:n]


# speedup vs baseline: 12.2868x; 12.2868x over previous
import functools

import jax
import jax.numpy as jnp
from jax import lax
from jax.experimental import pallas as pl
from jax.experimental.pallas import tpu as pltpu
from jax.experimental.pallas import tpu_sc as plsc

NC = 2
NS = 16
NT = NC * NS
CH = 128
RB = 512


def _sc_segment_sum(y, src2d, dst2d, n_pad):
    d = y.shape[1]
    n_chunks = src2d.shape[0]
    cpt = n_chunks // NT
    rpt = n_pad // NS

    mesh = plsc.VectorSubcoreMesh(core_axis_name="c", subcore_axis_name="s")

    @functools.partial(
        pl.kernel,
        out_type=jax.ShapeDtypeStruct((NC * n_pad, d), jnp.float32),
        mesh=mesh,
        compiler_params=pltpu.CompilerParams(use_tc_tiling_on_sc=False),
        scratch_types=[
            pltpu.VMEM((CH,), jnp.int32),
            pltpu.VMEM((CH,), jnp.int32),
            pltpu.VMEM((CH, d), jnp.float32),
            pltpu.VMEM_SHARED((n_pad, d), jnp.float32),
            pltpu.SemaphoreType.DMA,
        ],
    )
    def run(y_hbm, src_hbm, dst_hbm, out_hbm, sidx, didx, buf, accum, sem):
        c = lax.axis_index("c")
        s = lax.axis_index("s")
        wid = c * NS + s

        @pl.loop(0, CH)
        def _(r):
            @pl.loop(0, d, step=16)
            def _(col):
                buf[r, pl.ds(col, 16)] = jnp.zeros((16,), jnp.float32)

        @pl.loop(0, rpt, step=CH)
        def _(r):
            pltpu.sync_copy(buf, accum.at[pl.ds(s * rpt + r, CH)])

        plsc.subcore_barrier()

        @pl.loop(0, cpt)
        def _(i):
            ci = wid * cpt + i
            pltpu.sync_copy(src_hbm.at[ci], sidx)
            pltpu.sync_copy(dst_hbm.at[ci], didx)
            pltpu.async_copy(y_hbm.at[sidx], buf, sem).wait()
            pltpu.sync_copy(buf, accum.at[didx], add=True)

        plsc.subcore_barrier()
        pltpu.sync_copy(
            accum.at[pl.ds(s * rpt, rpt)],
            out_hbm.at[pl.ds(c * n_pad + s * rpt, rpt)],
        )

    return run(y, src2d, dst2d)


def _dinv(d0, d1):
    return lax.rsqrt(1.0 + d0[:, 0:1] + d1[:, 0:1])


def _mm_body(x_ref, w_ref, o_ref):
    o_ref[...] = jnp.dot(x_ref[...], w_ref[...],
                         preferred_element_type=jnp.float32,
                         precision=lax.Precision.HIGHEST)


def _tc_matmul(x_pad, w):
    n_pad, k = x_pad.shape
    m = w.shape[1]
    return pl.pallas_call(
        _mm_body,
        grid=(n_pad // RB,),
        in_specs=[pl.BlockSpec((RB, k), lambda i: (i, 0)),
                  pl.BlockSpec((k, m), lambda i: (0, 0))],
        out_specs=pl.BlockSpec((RB, m), lambda i: (i, 0)),
        out_shape=jax.ShapeDtypeStruct((n_pad, m), jnp.float32),
    )(x_pad, w)


def _scale_body(d0_ref, d1_ref, xw_ref, y_ref):
    y_ref[...] = xw_ref[...] * _dinv(d0_ref[...], d1_ref[...])


def _tc_scale(deg_p, xw):
    n_pad, dh = xw.shape
    nb = n_pad // RB
    return pl.pallas_call(
        _scale_body,
        grid=(nb,),
        in_specs=[pl.BlockSpec((RB, 16), lambda i: (i, 0)),
                  pl.BlockSpec((RB, 16), lambda i: (i + nb, 0)),
                  pl.BlockSpec((RB, dh), lambda i: (i, 0))],
        out_specs=pl.BlockSpec((RB, dh), lambda i: (i, 0)),
        out_shape=jax.ShapeDtypeStruct((n_pad, dh), jnp.float32),
    )(deg_p, deg_p, xw)


def _mid_body(d0, d1, a0, a1, xw1, b1, w2, xw2_o, y2_o):
    dinv = _dinv(d0[...], d1[...])
    h = (a0[...] + a1[...]) * dinv + xw1[...] * (dinv * dinv) + b1[...]
    h = jnp.maximum(h, 0.0)
    xw2 = jnp.dot(h, w2[...], preferred_element_type=jnp.float32,
                  precision=lax.Precision.HIGHEST)
    xw2_o[...] = xw2
    y2_o[...] = xw2 * dinv


def _tc_mid(deg_p, agg1_p, xw1, b1r, w2):
    n_pad, dh = xw1.shape
    do = w2.shape[1]
    nb = n_pad // RB
    return pl.pallas_call(
        _mid_body,
        grid=(nb,),
        in_specs=[pl.BlockSpec((RB, 16), lambda i: (i, 0)),
                  pl.BlockSpec((RB, 16), lambda i: (i + nb, 0)),
                  pl.BlockSpec((RB, dh), lambda i: (i, 0)),
                  pl.BlockSpec((RB, dh), lambda i: (i + nb, 0)),
                  pl.BlockSpec((RB, dh), lambda i: (i, 0)),
                  pl.BlockSpec((1, dh), lambda i: (0, 0)),
                  pl.BlockSpec((dh, do), lambda i: (0, 0))],
        out_specs=[pl.BlockSpec((RB, do), lambda i: (i, 0)),
                   pl.BlockSpec((RB, do), lambda i: (i, 0))],
        out_shape=[jax.ShapeDtypeStruct((n_pad, do), jnp.float32),
                   jax.ShapeDtypeStruct((n_pad, do), jnp.float32)],
    )(deg_p, deg_p, agg1_p, agg1_p, xw1, b1r, w2)


def _final_body(d0, d1, g0, g1, xw2, b2, o_ref):
    dinv = _dinv(d0[...], d1[...])
    o_ref[...] = (g0[...] + g1[...]) * dinv + xw2[...] * (dinv * dinv) + b2[...]


def _tc_final(deg_p, agg2_p, xw2, b2r):
    n_pad, do = xw2.shape
    nb = n_pad // RB
    return pl.pallas_call(
        _final_body,
        grid=(nb,),
        in_specs=[pl.BlockSpec((RB, 16), lambda i: (i, 0)),
                  pl.BlockSpec((RB, 16), lambda i: (i + nb, 0)),
                  pl.BlockSpec((RB, do), lambda i: (i, 0)),
                  pl.BlockSpec((RB, do), lambda i: (i + nb, 0)),
                  pl.BlockSpec((RB, do), lambda i: (i, 0)),
                  pl.BlockSpec((1, do), lambda i: (0, 0))],
        out_specs=pl.BlockSpec((RB, do), lambda i: (i, 0)),
        out_shape=jax.ShapeDtypeStruct((n_pad, do), jnp.float32),
    )(deg_p, deg_p, agg2_p, agg2_p, xw2, b2r)


def kernel(x, edge_index, W1, b1, W2, b2):
    n, d_in = x.shape
    e = edge_index.shape[1]

    blk = NS * CH
    n_pad = ((n + 1 + blk - 1) // blk) * blk
    epb = NT * CH
    e_pad = ((e + epb - 1) // epb) * epb

    x_pad = jnp.pad(x, ((0, n_pad - n), (0, 0)))
    pad_idx = jnp.full((e_pad - e,), n, dtype=jnp.int32)
    src2d = jnp.concatenate([edge_index[0], pad_idx]).reshape(e_pad // CH, CH)
    dst2d = jnp.concatenate([edge_index[1], pad_idx]).reshape(e_pad // CH, CH)

    ones16 = jnp.ones((n_pad, 16), jnp.float32)
    deg_p = _sc_segment_sum(ones16, dst2d, dst2d, n_pad)
    xw1 = _tc_matmul(x_pad, W1)
    y1 = _tc_scale(deg_p, xw1)
    agg1_p = _sc_segment_sum(y1, src2d, dst2d, n_pad)
    xw2, y2 = _tc_mid(deg_p, agg1_p, xw1, b1.reshape(1, -1), W2)
    agg2_p = _sc_segment_sum(y2, src2d, dst2d, n_pad)
    out = _tc_final(deg_p, agg2_p, xw2, b2.reshape(1, -1))
    return out[:n]
